# Initial kernel scaffold; baseline (speedup 1.0000x reference)
import jax
import jax.numpy as jnp
from jax.experimental import pallas as pl


def kernel(x, fc1_w, fc1_b, fc2_w, fc2_b, fc3_w, fc3_b, fc4_w, fc4_b, emb_w):
    raise NotImplementedError("write your pallas kernel here")


# speedup vs baseline: 2.4077x; 2.4077x over previous
import functools

import jax
import jax.numpy as jnp
from jax.experimental import pallas as pl

INPUT_SIZE = 3072
HIDDEN = 512
K = 64
L = HIDDEN // K
BATCH = 2048
TB = 256


def _fused_kernel(x_ref, w1t_ref, b1_ref, w2t_ref, b2_ref, embe_ref, embt_ref,
                  w3t_ref, b3_ref, w4t_ref, b4_ref,
                  recon_ref, zef_ref, zqf_ref):
    f32 = jnp.float32
    x = x_ref[...]
    h1 = jnp.maximum(jnp.dot(x, w1t_ref[...], preferred_element_type=f32)
                     + b1_ref[...], 0.0)
    h2 = jnp.dot(h1, w2t_ref[...], preferred_element_type=f32) + b2_ref[...]
    zef_ref[...] = h2

    embe = embe_ref[...]
    scores = jnp.dot(h2, embe, preferred_element_type=f32,
                 precision=jax.lax.Precision.HIGHEST)
    csq = jnp.sum(embe * embe, axis=0, keepdims=True)
    pen = csq - 2.0 * scores

    embt = embt_ref[...]
    parts = []
    for l in range(L):
        p = pen[:, l * K:(l + 1) * K]
        am = jnp.argmin(p, axis=1)
        iota = jax.lax.broadcasted_iota(jnp.int32, (TB, K), 1)
        oh = (am[:, None] == iota).astype(f32)
        parts.append(jnp.dot(oh, embt, preferred_element_type=f32,
                 precision=jax.lax.Precision.HIGHEST))
    zq = jnp.concatenate(parts, axis=1)
    zqf_ref[...] = zq

    h3 = jnp.maximum(jnp.dot(zq, w3t_ref[...], preferred_element_type=f32)
                     + b3_ref[...], 0.0)
    recon_ref[...] = jnp.tanh(
        jnp.dot(h3, w4t_ref[...], preferred_element_type=f32) + b4_ref[...])


@functools.partial(jax.jit, static_argnames=("interpret",))
def _run(x, fc1_w, fc1_b, fc2_w, fc2_b, fc3_w, fc3_b, fc4_w, fc4_b, emb_w,
         interpret=False):
    w1t = fc1_w.T
    w2t = fc2_w.T
    w3t = fc3_w.reshape(400, K, L).transpose(0, 2, 1).reshape(400, HIDDEN).T
    w4t = fc4_w.T
    b1 = fc1_b.reshape(1, -1)
    b2 = fc2_b.reshape(1, -1)
    b3 = fc3_b.reshape(1, -1)
    b4 = fc4_b.reshape(1, -1)
    eyeL = jnp.eye(L, dtype=emb_w.dtype)
    embe = jnp.einsum('kj,lm->klmj', emb_w, eyeL).reshape(HIDDEN, HIDDEN)
    embt = emb_w.T

    const = lambda shape: pl.BlockSpec(shape, lambda i: (0, 0))
    recon, zef, zqf = pl.pallas_call(
        _fused_kernel,
        grid=(BATCH // TB,),
        in_specs=[
            pl.BlockSpec((TB, INPUT_SIZE), lambda i: (i, 0)),
            const((INPUT_SIZE, 400)), const((1, 400)),
            const((400, HIDDEN)), const((1, HIDDEN)),
            const((HIDDEN, HIDDEN)), const((K, K)),
            const((HIDDEN, 400)), const((1, 400)),
            const((400, INPUT_SIZE)), const((1, INPUT_SIZE)),
        ],
        out_specs=[
            pl.BlockSpec((TB, INPUT_SIZE), lambda i: (i, 0)),
            pl.BlockSpec((TB, HIDDEN), lambda i: (i, 0)),
            pl.BlockSpec((TB, HIDDEN), lambda i: (i, 0)),
        ],
        out_shape=[
            jax.ShapeDtypeStruct((BATCH, INPUT_SIZE), jnp.float32),
            jax.ShapeDtypeStruct((BATCH, HIDDEN), jnp.float32),
            jax.ShapeDtypeStruct((BATCH, HIDDEN), jnp.float32),
        ],
        interpret=interpret,
    )(x, w1t, b1, w2t, b2, embe, embt, w3t, b3, w4t, b4)

    z_e = zef.reshape(BATCH, K, L)
    emb_o = zqf.reshape(BATCH, L, K).transpose(0, 2, 1)
    return recon, z_e, emb_o


def kernel(x, fc1_w, fc1_b, fc2_w, fc2_b, fc3_w, fc3_b, fc4_w, fc4_b, emb_w):
    return _run(x, fc1_w, fc1_b, fc2_w, fc2_b, fc3_w, fc3_b, fc4_w, fc4_b,
                emb_w)
